# baseline (speedup 1.0000x reference)
import functools

import jax
import jax.numpy as jnp
from jax import lax
from jax.experimental import pallas as pl
from jax.experimental.pallas import tpu as pltpu
from jax.experimental.pallas import tpu_sc as plsc

VOCAB = 100000
EMBED_DIM = 1024
BATCH = 16384

_info = plsc.get_sparse_core_info()
_NC, _NS = _info.num_cores, _info.num_subcores
NW = _NC * _NS
B_PER_W = BATCH // NW
CHUNK = 32
N_CHUNKS = B_PER_W // CHUNK
NBUF = 3
LOOP_ITERS = 4


@functools.partial(
    pl.kernel,
    mesh=plsc.VectorSubcoreMesh(core_axis_name="c", subcore_axis_name="s"),
    out_type=jax.ShapeDtypeStruct((BATCH, EMBED_DIM), jnp.float32),
    scratch_types=[
        pltpu.VMEM((B_PER_W,), jnp.int32),
        *[pltpu.VMEM((CHUNK, EMBED_DIM), jnp.float32) for _ in range(NBUF)],
        *[pltpu.SemaphoreType.DMA for _ in range(2 * NBUF)],
    ],
)
def _gather_kernel(idx_hbm, table_hbm, out_hbm, idx_v, *bufs_and_sems):
    bufs = bufs_and_sems[:NBUF]
    gsems = bufs_and_sems[NBUF:2 * NBUF]
    wsems = bufs_and_sems[2 * NBUF:]
    wid = lax.axis_index("s") * _NC + lax.axis_index("c")
    base = wid * B_PER_W
    pltpu.sync_copy(idx_hbm.at[pl.ds(base, B_PER_W)], idx_v)

    def gather_copy(i, b):
        return pltpu.make_async_copy(
            table_hbm.at[idx_v.at[pl.ds(i * CHUNK, CHUNK)]], bufs[b],
            gsems[b])

    def write_copy(i, b):
        return pltpu.make_async_copy(
            bufs[b], out_hbm.at[pl.ds(base + i * CHUNK, CHUNK)], wsems[b])

    gather_copy(0, 0).start()
    gather_copy(1, 1).start()
    gather_copy(2, 2).start()
    gather_copy(0, 0).wait()
    write_copy(0, 0).start()

    def body(g, carry):
        i0 = 1 + 3 * g
        for k in range(3):
            i = i0 + k
            b = (1 + k) % 3
            bn = k
            gather_copy(i, b).wait()
            write_copy(i, b).start()
            write_copy(i - 1, bn).wait()
            gather_copy(i + 2, bn).start()
        return carry

    lax.fori_loop(0, LOOP_ITERS, body, 0)

    gather_copy(13, 1).wait()
    write_copy(13, 1).start()
    write_copy(12, 0).wait()
    gather_copy(15, 0).start()
    gather_copy(14, 2).wait()
    write_copy(14, 2).start()
    gather_copy(15, 0).wait()
    write_copy(15, 0).start()
    write_copy(13, 1).wait()
    write_copy(14, 2).wait()
    write_copy(15, 0).wait()


def kernel(indices, table):
    return _gather_kernel(indices, table)
